# baseline (speedup 1.0000x reference)
import functools

import jax
import jax.numpy as jnp
from jax import lax
from jax.experimental import pallas as pl
from jax.experimental.pallas import tpu as pltpu
from jax.experimental.pallas import tpu_sc as plsc

_POOL_ROWS = 1000000
_DIM = 64
_BATCH = 16384

_NUM_CORES = 2
_NUM_SUBCORES = 16
_NUM_WORKERS = _NUM_CORES * _NUM_SUBCORES
_B_PER_W = _BATCH // _NUM_WORKERS

_BLK = 4096
_GRID = -(-_POOL_ROWS // (2 * _BLK))
_PACK_ROWS = _GRID * _BLK

_mesh = plsc.VectorSubcoreMesh(core_axis_name="c", subcore_axis_name="s")


def _repack_body(cols_ref, out_ref):
    x = cols_ref[...]
    ident = (lax.broadcasted_iota(jnp.int32, (_DIM, _DIM), 0)
             == lax.broadcasted_iota(jnp.int32, (_DIM, _DIM), 1)).astype(jnp.float32)
    y = lax.dot_general(
        x, ident, dimension_numbers=(((0,), (0,)), ((), ())),
        preferred_element_type=jnp.float32)
    out_ref[...] = jnp.concatenate([y[:_BLK], y[_BLK:]], axis=1)


def _repack(shard_t):
    return pl.pallas_call(
        _repack_body,
        grid=(_GRID,),
        in_specs=[pl.BlockSpec((_DIM, 2 * _BLK), lambda j: (0, j))],
        out_specs=pl.BlockSpec((_BLK, 2 * _DIM), lambda j: (j, 0)),
        out_shape=jax.ShapeDtypeStruct((_PACK_ROWS, 2 * _DIM), jnp.float32),
    )(shard_t)


@functools.partial(
    pl.kernel,
    mesh=_mesh,
    out_type=jax.ShapeDtypeStruct((_BATCH, 2 * _DIM), jnp.float32),
    scratch_types=[
        pltpu.VMEM((_B_PER_W,), jnp.int32),
        pltpu.VMEM((_B_PER_W, 2 * _DIM), jnp.float32),
        pltpu.SemaphoreType.DMA,
    ],
)
def _sc_gather_pairs(idx2_hbm, pairs_hbm, out_hbm, idx_v, rows_v, sem):
    wid = lax.axis_index("s") * _NUM_CORES + lax.axis_index("c")
    base = wid * _B_PER_W
    pltpu.sync_copy(idx2_hbm.at[pl.ds(base, _B_PER_W)], idx_v)
    pltpu.async_copy(pairs_hbm.at[idx_v], rows_v, sem).wait()
    pltpu.sync_copy(rows_v, out_hbm.at[pl.ds(base, _B_PER_W)])


def kernel(rank_ids, shard):
    idx = rank_ids.astype(jnp.int32)
    pairs = _repack(shard.T)
    pair_row = ((idx >> 13) << 12) | (idx & (_BLK - 1))
    gathered = _sc_gather_pairs(pair_row, pairs)
    right = (idx & _BLK).astype(bool)
    return jnp.where(right[:, None], gathered[:, _DIM:], gathered[:, :_DIM])


# speedup vs baseline: 1.1945x; 1.1945x over previous
import functools

import jax
import jax.numpy as jnp
from jax import lax
from jax.experimental import pallas as pl
from jax.experimental.pallas import tpu as pltpu
from jax.experimental.pallas import tpu_sc as plsc

_POOL_ROWS = 1000000
_DIM = 64
_BATCH = 16384

_NUM_CORES = 2
_NUM_SUBCORES = 16
_NUM_WORKERS = _NUM_CORES * _NUM_SUBCORES
_B_PER_W = _BATCH // _NUM_WORKERS

_BLK = 16384
_GRID = -(-_POOL_ROWS // (2 * _BLK))
_PACK_ROWS = _GRID * _BLK

_mesh = plsc.VectorSubcoreMesh(core_axis_name="c", subcore_axis_name="s")


def _repack_body(cols_ref, out_ref):
    x = cols_ref[...]
    out_ref[...] = jnp.concatenate([x[:, :_BLK].T, x[:, _BLK:].T], axis=1)


def _repack(shard_t):
    return pl.pallas_call(
        _repack_body,
        grid=(_GRID,),
        in_specs=[pl.BlockSpec((_DIM, 2 * _BLK), lambda j: (0, j))],
        out_specs=pl.BlockSpec((_BLK, 2 * _DIM), lambda j: (j, 0)),
        out_shape=jax.ShapeDtypeStruct((_PACK_ROWS, 2 * _DIM), jnp.float32),
        compiler_params=pltpu.CompilerParams(
            dimension_semantics=("parallel",)),
    )(shard_t)


@functools.partial(
    pl.kernel,
    mesh=_mesh,
    out_type=jax.ShapeDtypeStruct((_BATCH, 2 * _DIM), jnp.float32),
    scratch_types=[
        pltpu.VMEM((_B_PER_W,), jnp.int32),
        pltpu.VMEM((_B_PER_W, 2 * _DIM), jnp.float32),
        pltpu.SemaphoreType.DMA,
    ],
)
def _sc_gather_pairs(idx2_hbm, pairs_hbm, out_hbm, idx_v, rows_v, sem):
    wid = lax.axis_index("s") * _NUM_CORES + lax.axis_index("c")
    base = wid * _B_PER_W
    pltpu.sync_copy(idx2_hbm.at[pl.ds(base, _B_PER_W)], idx_v)
    pltpu.async_copy(pairs_hbm.at[idx_v], rows_v, sem).wait()
    pltpu.sync_copy(rows_v, out_hbm.at[pl.ds(base, _B_PER_W)])


def kernel(rank_ids, shard):
    idx = rank_ids.astype(jnp.int32)
    pairs = _repack(shard.T)
    pair_row = ((idx >> 15) << 14) | (idx & (_BLK - 1))
    gathered = _sc_gather_pairs(pair_row, pairs)
    right = (idx & _BLK).astype(bool)
    return jnp.where(right[:, None], gathered[:, _DIM:], gathered[:, :_DIM])
